# baseline (speedup 1.0000x reference)
import functools

import jax
import jax.numpy as jnp
from jax import lax
from jax.experimental import pallas as pl
from jax.experimental.pallas import tpu as pltpu
from jax.experimental.pallas import tpu_sc as plsc

NC = 2
NS = 16
NW = NC * NS
K = 128
L = 16


def _sc_agg_call(n_pad, spw, d):
    mesh = plsc.VectorSubcoreMesh(core_axis_name="c", subcore_axis_name="s")
    rows_per_tile = n_pad // NS

    @functools.partial(
        pl.kernel,
        mesh=mesh,
        out_type=jax.ShapeDtypeStruct((NC, n_pad, d), jnp.float32),
        scratch_types=[
            pltpu.VMEM((spw, K), jnp.int32),
            pltpu.VMEM((K,), jnp.int32),
            pltpu.VMEM((K,), jnp.int32),
            pltpu.VMEM((K,), jnp.int32),
            pltpu.VMEM((K,), jnp.int32),
            pltpu.VMEM((K, d), jnp.float32),
            pltpu.VMEM((K, d), jnp.float32),
            pltpu.VMEM_SHARED((n_pad, d), jnp.float32),
            pltpu.SemaphoreType.DMA,
            pltpu.SemaphoreType.DMA,
            pltpu.SemaphoreType.DMA,
        ],
    )
    def sc_agg(x_hbm, pk_hbm, zeros_hbm, out_hbm,
               pk_v, src0, src1, dst0, dst1, rows0, rows1, agg_sh,
               sem0, sem1, isem):
        c = lax.axis_index("c")
        s = lax.axis_index("s")
        wid = c * NS + s
        srcs = (src0, src1)
        dsts = (dst0, dst1)
        rows = (rows0, rows1)
        sems = (sem0, sem1)

        idx_cp = pltpu.async_copy(pk_hbm.at[pl.ds(wid * spw, spw)], pk_v, isem)
        zcp = pltpu.async_copy(
            zeros_hbm.at[pl.ds(s * rows_per_tile, rows_per_tile)],
            agg_sh.at[pl.ds(s * rows_per_tile, rows_per_tile)],
            sem0,
        )
        idx_cp.wait()
        zcp.wait()
        plsc.subcore_barrier()

        def unpack(g, b):
            for j in range(K // L):
                v = pk_v[g, pl.ds(j * L, L)]
                srcs[b][pl.ds(j * L, L)] = lax.bitwise_and(v, 0xFFFF)
                dsts[b][pl.ds(j * L, L)] = lax.shift_right_logical(v, 16)

        def gather_start(b):
            pltpu.async_copy(x_hbm.at[srcs[b]], rows[b], sems[b])

        def gather_wait(b):
            pltpu.make_async_copy(x_hbm.at[srcs[b]], rows[b], sems[b]).wait()

        unpack(0, 0)
        gather_start(0)
        unpack(1, 1)
        gather_start(1)

        def step(i, carry):
            g0 = i * 2
            for b in range(2):
                g = g0 + b
                gather_wait(b)
                pltpu.sync_copy(rows[b], agg_sh.at[dsts[b]], add=True)

                @pl.when(g + 2 < spw)
                def _():
                    unpack(g + 2, b)
                    gather_start(b)
            return carry

        lax.fori_loop(0, spw // 2, step, 0)
        plsc.subcore_barrier()
        pltpu.sync_copy(
            agg_sh.at[pl.ds(s * rows_per_tile, rows_per_tile)],
            out_hbm.at[c, pl.ds(s * rows_per_tile, rows_per_tile)],
        )

    return sc_agg


def _dense_body(n, xr, p0r, p1r, w1r, b1r, w2r, b2r, gr, br, outr):
    h = xr[...] + p0r[...][:n] + p1r[...][:n]
    a = jnp.dot(h, w1r[...], preferred_element_type=jnp.float32) + b1r[...]
    a = jnp.maximum(a, 0.0)
    h2 = jnp.dot(a, w2r[...], preferred_element_type=jnp.float32) + b2r[...]
    mean = jnp.mean(h2, axis=0, keepdims=True)
    cent = h2 - mean
    var = jnp.mean(cent * cent, axis=0, keepdims=True)
    scale = lax.rsqrt(var + 1e-5) * gr[...]
    outr[...] = jnp.maximum(cent * scale + br[...], 0.0)


def kernel(x, edge_index, W1, b1, W2, b2, gamma, beta):
    n, d = x.shape
    e = edge_index.shape[1]
    spw = -(-e // (NW * K))
    spw += spw % 2
    e_pad = spw * K * NW
    n_pad = -(-(n + 1) // (NS * 8)) * (NS * 8)
    dummy = n_pad - 1

    src = edge_index[0].astype(jnp.int32)
    dst = edge_index[1].astype(jnp.int32)
    packed = jnp.bitwise_or(src, jnp.left_shift(dst, 16))
    packed = jnp.concatenate(
        [packed, jnp.full((e_pad - e,), dummy << 16, jnp.int32)])
    packed = packed.reshape(NW * spw, K)
    zeros = jnp.zeros((n_pad, d), jnp.float32)

    partials = _sc_agg_call(n_pad, spw, d)(x, packed, zeros)

    out = pl.pallas_call(
        functools.partial(_dense_body, n),
        out_shape=jax.ShapeDtypeStruct((n, d), jnp.float32),
    )(x, partials[0], partials[1], W1.T, b1.reshape(1, d), W2.T,
      b2.reshape(1, d), gamma.reshape(1, d), beta.reshape(1, d))
    return out
